# Initial kernel scaffold; baseline (speedup 1.0000x reference)
import jax
import jax.numpy as jnp
from jax.experimental import pallas as pl


def kernel(input, token_table, segment_table, pe):
    raise NotImplementedError("write your pallas kernel here")


# speedup vs baseline: 1.8806x; 1.8806x over previous
import functools

import jax
import jax.numpy as jnp
from jax import lax
from jax.experimental import pallas as pl
from jax.experimental.pallas import tpu as pltpu
from jax.experimental.pallas import tpu_sc as plsc

B = 1024
L = 200
D = 128
VOCAB = 100000
N_SEG = 3

NC, NS = 2, 16
NW = NC * NS
TOT = B * L
ROWS_W = TOT // NW
CHUNK = 128
N_CHUNKS = ROWS_W // CHUNK
IDX_ROWS = TOT // CHUNK
IDX_W = IDX_ROWS // NW


def _sc_embed(token_table, comb_table, tok_idx, comb_idx):
    mesh = plsc.VectorSubcoreMesh(core_axis_name="c", subcore_axis_name="s")

    @functools.partial(
        pl.kernel,
        out_type=jax.ShapeDtypeStruct((TOT, D), jnp.float32),
        mesh=mesh,
        scratch_types=[
            pltpu.VMEM((N_SEG * L, D), jnp.float32),
            pltpu.VMEM((IDX_W, CHUNK), jnp.int32),
            pltpu.VMEM((IDX_W, CHUNK), jnp.int32),
            pltpu.VMEM((CHUNK, D), jnp.float32),
            pltpu.SemaphoreType.DMA,
        ],
    )
    def k(tok_hbm, comb_hbm, tidx_hbm, cidx_hbm, out_hbm,
          comb_v, tidx_v, cidx_v, rows_v, sem):
        wid = lax.axis_index("s") * NC + lax.axis_index("c")
        pltpu.sync_copy(comb_hbm, comb_v)
        pltpu.sync_copy(tidx_hbm.at[wid], tidx_v)
        pltpu.sync_copy(cidx_hbm.at[wid], cidx_v)
        base = wid * ROWS_W

        def chunk_body(kk, _):
            pltpu.async_copy(tok_hbm.at[tidx_v.at[kk]], rows_v, sem).wait()

            def row_grp(q, _):
                civ = cidx_v[kk, pl.ds(16 * q, 16)]
                for j in range(16):
                    ci = civ[j]
                    r = 16 * q + j
                    for g in range(D // 16):
                        val = comb_v[ci, pl.ds(16 * g, 16)]
                        plsc.addupdate(rows_v.at[r, pl.ds(16 * g, 16)], val)
                return 0

            lax.fori_loop(0, CHUNK // 16, row_grp, 0)
            pltpu.sync_copy(rows_v, out_hbm.at[pl.ds(base + kk * CHUNK, CHUNK)])
            return 0

        lax.fori_loop(0, N_CHUNKS, chunk_body, 0)

    return k(token_table, comb_table, tok_idx, comb_idx)


def _mask_body(seq_ref, out_ref):
    m = seq_ref[...] > 0
    out_ref[...] = jnp.broadcast_to(m[:, None, None, :], out_ref.shape)


def _tc_mask(seq):
    Bb = 32
    return pl.pallas_call(
        _mask_body,
        grid=(B // Bb,),
        in_specs=[pl.BlockSpec((Bb, L), lambda i: (i, 0))],
        out_specs=pl.BlockSpec((Bb, 1, L, L), lambda i: (i, 0, 0, 0)),
        out_shape=jax.ShapeDtypeStruct((B, 1, L, L), jnp.bool_),
    )(seq)


def kernel(input, token_table, segment_table, pe):
    seq = input[0]
    seg = input[1]
    comb = (pe[0, :L, None, :] + segment_table[None, :, :]).reshape(N_SEG * L, D)
    tok_idx = seq.reshape(NW, IDX_W, CHUNK)
    comb_idx = (jnp.arange(L, dtype=jnp.int32)[None, :] * N_SEG + seg).reshape(
        NW, IDX_W, CHUNK)
    x = _sc_embed(token_table, comb, tok_idx, comb_idx).reshape(B, L, D)
    mask = _tc_mask(seq)
    return (x, mask)


# speedup vs baseline: 4.5097x; 2.3980x over previous
import functools

import jax
import jax.numpy as jnp
from jax import lax
from jax.experimental import pallas as pl
from jax.experimental.pallas import tpu as pltpu
from jax.experimental.pallas import tpu_sc as plsc

B = 1024
L = 200
D = 128
N_SEG = 3

NC, NS = 2, 16
NW = NC * NS
TOT = B * L
ROWS_W = TOT // NW
CH = 128
NCHUNK = ROWS_W // CH
NBUF = 4


def _sc_embed(token_table, pe2d, seg_table, tok_idx, seg_idx):
    mesh = plsc.VectorSubcoreMesh(core_axis_name="c", subcore_axis_name="s")

    @functools.partial(
        pl.kernel,
        out_type=jax.ShapeDtypeStruct((TOT, D), jnp.float32),
        mesh=mesh,
        scratch_types=[
            pltpu.VMEM((L, D), jnp.float32),
            pltpu.VMEM((N_SEG, D), jnp.float32),
            pltpu.VMEM((NCHUNK, CH), jnp.int32),
            pltpu.VMEM((NCHUNK, CH), jnp.int32),
        ] + [pltpu.VMEM((CH, D), jnp.float32)] * NBUF
          + [pltpu.SemaphoreType.DMA] * (2 * NBUF),
    )
    def k(tok_hbm, pe_hbm, seg_hbm, tidx_hbm, sidx_hbm, out_hbm,
          pe_v, seg_v, tidx_v, sidx_v, buf0, buf1, buf2, buf3,
          sg0, sg1, sg2, sg3, ss0, ss1, ss2, ss3):
        bufs = (buf0, buf1, buf2, buf3)
        sgs = (sg0, sg1, sg2, sg3)
        sss = (ss0, ss1, ss2, ss3)
        wid = lax.axis_index("s") * NC + lax.axis_index("c")
        pltpu.sync_copy(pe_hbm, pe_v)
        pltpu.sync_copy(seg_hbm, seg_v)
        pltpu.sync_copy(tidx_hbm.at[wid], tidx_v)
        pltpu.sync_copy(sidx_hbm.at[wid], sidx_v)
        base = wid * ROWS_W

        def gather(c, b):
            pltpu.async_copy(tok_hbm.at[tidx_v.at[c]], bufs[b], sgs[b])

        def swait(c, b):
            pltpu.make_async_copy(
                bufs[b], out_hbm.at[pl.ds(base + c * CH, CH)], sss[b]).wait()

        def chunk(c, b, prefetch):
            bn = (b + 2) % NBUF
            if prefetch == "first":
                gather(c + 2, bn)
            elif prefetch == "steady":
                swait(c - 2, bn)
                gather(c + 2, bn)
            pltpu.make_async_copy(
                tok_hbm.at[tidx_v.at[c]], bufs[b], sgs[b]).wait()

            def row_grp(q, _):
                siv = sidx_v[c, pl.ds(16 * q, 16)]
                lbase = c * CH + 16 * q
                for j in range(16):
                    sj = siv[j]
                    lj = lax.rem(lbase + j, L)
                    r = 16 * q + j
                    for g in range(D // 16):
                        val = (pe_v[lj, pl.ds(16 * g, 16)]
                               + seg_v[sj, pl.ds(16 * g, 16)])
                        plsc.addupdate(bufs[b].at[r, pl.ds(16 * g, 16)], val)
                return 0

            lax.fori_loop(0, CH // 16, row_grp, 0)
            pltpu.async_copy(
                bufs[b], out_hbm.at[pl.ds(base + c * CH, CH)], sss[b])

        gather(0, 0)
        gather(1, 1)
        chunk(0, 0, "first")
        chunk(1, 1, "first")

        def quad_body(u, _):
            for j in range(NBUF):
                chunk(NBUF * u + 2 + j, (2 + j) % NBUF, "steady")
            return 0

        lax.fori_loop(0, (NCHUNK - 6) // NBUF, quad_body, 0)
        chunk(NCHUNK - 4, 2, "steady")
        chunk(NCHUNK - 3, 3, "steady")
        chunk(NCHUNK - 2, 0, "tail")
        chunk(NCHUNK - 1, 1, "tail")
        for c in range(NCHUNK - 4, NCHUNK):
            swait(c, c % NBUF)

    return k(token_table, pe2d, seg_table, tok_idx, seg_idx)


def _mask_body(seqt_ref, out_ref):
    m = (seqt_ref[...] > 0).astype(jnp.int8)
    out_ref[...] = jnp.broadcast_to(m[None, None, :, :], out_ref.shape)


def _tc_mask(seqt):
    Ib = 25
    maskt = pl.pallas_call(
        _mask_body,
        grid=(L // Ib,),
        in_specs=[pl.BlockSpec((L, B), lambda i: (0, 0))],
        out_specs=pl.BlockSpec((1, Ib, L, B), lambda i: (0, i, 0, 0)),
        out_shape=jax.ShapeDtypeStruct((1, L, L, B), jnp.int8),
    )(seqt)
    return jnp.transpose(maskt, (3, 0, 1, 2)).astype(jnp.bool_)


def kernel(input, token_table, segment_table, pe):
    seq = input[0]
    seg = input[1]
    tok_idx = seq.reshape(NW, NCHUNK, CH)
    seg_idx = seg.reshape(NW, NCHUNK, CH)
    x = _sc_embed(token_table, pe[0, :L], segment_table, tok_idx,
                  seg_idx).reshape(B, L, D)
    mask = _tc_mask(seq.T)
    return (x, mask)


# speedup vs baseline: 9.1380x; 2.0263x over previous
import functools

import jax
import jax.numpy as jnp
from jax import lax
from jax.experimental import pallas as pl
from jax.experimental.pallas import tpu as pltpu
from jax.experimental.pallas import tpu_sc as plsc

B = 1024
L = 200
D = 128
N_SEG = 3

NC, NS = 2, 16
NW = NC * NS
TOT = B * L
ROWS_W = TOT // NW
CH = 128
NCHUNK = ROWS_W // CH
NBUF = 4


def _sc_embed(token_table, pe2d, seg_table, tok_idx, seg_idx):
    mesh = plsc.VectorSubcoreMesh(core_axis_name="c", subcore_axis_name="s")

    @functools.partial(
        pl.kernel,
        out_type=jax.ShapeDtypeStruct((TOT, D), jnp.float32),
        mesh=mesh,
        scratch_types=[
            pltpu.VMEM((L, D), jnp.float32),
            pltpu.VMEM((N_SEG, D), jnp.float32),
            pltpu.VMEM((NCHUNK, CH), jnp.int32),
            pltpu.VMEM((NCHUNK, CH), jnp.int32),
        ] + [pltpu.VMEM((CH, D), jnp.float32)] * NBUF
          + [pltpu.SemaphoreType.DMA] * (2 * NBUF),
    )
    def k(tok_hbm, pe_hbm, seg_hbm, tidx_hbm, sidx_hbm, out_hbm,
          pe_v, seg_v, tidx_v, sidx_v, buf0, buf1, buf2, buf3,
          sg0, sg1, sg2, sg3, ss0, ss1, ss2, ss3):
        bufs = (buf0, buf1, buf2, buf3)
        sgs = (sg0, sg1, sg2, sg3)
        sss = (ss0, ss1, ss2, ss3)
        wid = lax.axis_index("s") * NC + lax.axis_index("c")
        pltpu.sync_copy(pe_hbm, pe_v)
        pltpu.sync_copy(seg_hbm, seg_v)
        pltpu.sync_copy(tidx_hbm.at[wid], tidx_v)
        pltpu.sync_copy(sidx_hbm.at[wid], sidx_v)
        base = wid * ROWS_W

        def gather(c, b):
            pltpu.async_copy(tok_hbm.at[tidx_v.at[c]], bufs[b], sgs[b])

        def swait(c, b):
            pltpu.make_async_copy(
                bufs[b], out_hbm.at[pl.ds(base + c * CH, CH)], sss[b]).wait()

        def chunk(c, b, prefetch):
            bn = (b + 2) % NBUF
            if prefetch == "first":
                gather(c + 2, bn)
            elif prefetch == "steady":
                swait(c - 2, bn)
                gather(c + 2, bn)
            pltpu.make_async_copy(
                tok_hbm.at[tidx_v.at[c]], bufs[b], sgs[b]).wait()

            def row_grp(q, _):
                siv = sidx_v[c, pl.ds(16 * q, 16)]
                l0 = lax.rem(c * CH + 16 * q, L)
                for j in range(16):
                    sj = siv[j]
                    w = l0 + j
                    lj = jnp.where(w >= L, w - L, w)
                    r = 16 * q + j
                    vals = [pe_v[lj, pl.ds(16 * g, 16)]
                            + seg_v[sj, pl.ds(16 * g, 16)]
                            for g in range(D // 16)]
                    for g in range(D // 16):
                        plsc.addupdate(bufs[b].at[r, pl.ds(16 * g, 16)], vals[g])
                return 0

            lax.fori_loop(0, CH // 16, row_grp, 0)
            pltpu.async_copy(
                bufs[b], out_hbm.at[pl.ds(base + c * CH, CH)], sss[b])

        gather(0, 0)
        gather(1, 1)
        chunk(0, 0, "first")
        chunk(1, 1, "first")

        def quad_body(u, _):
            for j in range(NBUF):
                chunk(NBUF * u + 2 + j, (2 + j) % NBUF, "steady")
            return 0

        lax.fori_loop(0, (NCHUNK - 6) // NBUF, quad_body, 0)
        chunk(NCHUNK - 4, 2, "steady")
        chunk(NCHUNK - 3, 3, "steady")
        chunk(NCHUNK - 2, 0, "tail")
        chunk(NCHUNK - 1, 1, "tail")
        for c in range(NCHUNK - 4, NCHUNK):
            swait(c, c % NBUF)

    return k(token_table, pe2d, seg_table, tok_idx, seg_idx)


def _mask_body(seqt_ref, out_ref):
    m = (seqt_ref[...] > 0).astype(jnp.int8)
    out_ref[...] = jnp.broadcast_to(m[None, None, :, :], out_ref.shape)


def _tc_mask(seqt):
    Ib = 25
    maskt = pl.pallas_call(
        _mask_body,
        grid=(L // Ib,),
        in_specs=[pl.BlockSpec((L, B), lambda i: (0, 0))],
        out_specs=pl.BlockSpec((1, Ib, L, B), lambda i: (0, i, 0, 0)),
        out_shape=jax.ShapeDtypeStruct((1, L, L, B), jnp.int8),
    )(seqt)
    return jnp.transpose(maskt, (3, 0, 1, 2)).astype(jnp.bool_)


def kernel(input, token_table, segment_table, pe):
    seq = input[0]
    seg = input[1]
    tok_idx = seq.reshape(NW, NCHUNK, CH)
    seg_idx = seg.reshape(NW, NCHUNK, CH)
    x = _sc_embed(token_table, pe[0, :L], segment_table, tok_idx,
                  seg_idx).reshape(B, L, D)
    mask = _tc_mask(seq.T)
    return (x, mask)


# speedup vs baseline: 10.0637x; 1.1013x over previous
import functools

import jax
import jax.numpy as jnp
from jax import lax
from jax.experimental import pallas as pl
from jax.experimental.pallas import tpu as pltpu
from jax.experimental.pallas import tpu_sc as plsc

B = 1024
L = 200
D = 128
N_SEG = 3

NC, NS = 2, 16
NW = NC * NS
TOT = B * L
ROWS_W = TOT // NW
CH = 128
NCHUNK = ROWS_W // CH
NBUF = 4


def _sc_embed(token_table, pe2d, seg_table, tok_idx, seg_idx):
    mesh = plsc.VectorSubcoreMesh(core_axis_name="c", subcore_axis_name="s")

    @functools.partial(
        pl.kernel,
        out_type=jax.ShapeDtypeStruct((TOT, D), jnp.float32),
        mesh=mesh,
        scratch_types=[
            pltpu.VMEM((L, D), jnp.float32),
            pltpu.VMEM((N_SEG, D), jnp.float32),
            pltpu.VMEM((NCHUNK, CH), jnp.int32),
            pltpu.VMEM((NCHUNK, CH), jnp.int32),
        ] + [pltpu.VMEM((CH, D), jnp.float32)] * NBUF
          + [pltpu.SemaphoreType.DMA] * (2 * NBUF),
    )
    def k(tok_hbm, pe_hbm, seg_hbm, tidx_hbm, sidx_hbm, out_hbm,
          pe_v, seg_v, tidx_v, sidx_v, buf0, buf1, buf2, buf3,
          sg0, sg1, sg2, sg3, ss0, ss1, ss2, ss3):
        bufs = (buf0, buf1, buf2, buf3)
        sgs = (sg0, sg1, sg2, sg3)
        sss = (ss0, ss1, ss2, ss3)
        wid = lax.axis_index("s") * NC + lax.axis_index("c")
        pltpu.sync_copy(pe_hbm, pe_v)
        pltpu.sync_copy(seg_hbm, seg_v)
        pltpu.sync_copy(tidx_hbm.at[wid], tidx_v)
        pltpu.sync_copy(sidx_hbm.at[wid], sidx_v)
        base = wid * ROWS_W

        def gather(c, b):
            pltpu.async_copy(tok_hbm.at[tidx_v.at[c]], bufs[b], sgs[b])

        def swait(c, b):
            pltpu.make_async_copy(
                bufs[b], out_hbm.at[pl.ds(base + c * CH, CH)], sss[b]).wait()

        def chunk(c, b, prefetch):
            bn = (b + 2) % NBUF
            if prefetch == "first":
                gather(c + 2, bn)
            elif prefetch == "steady":
                swait(c - 2, bn)
                gather(c + 2, bn)
            pltpu.make_async_copy(
                tok_hbm.at[tidx_v.at[c]], bufs[b], sgs[b]).wait()

            def row_grp(q, _):
                siv = sidx_v[c, pl.ds(16 * q, 16)]
                l0 = lax.rem(c * CH + 16 * q, L)
                sgv = [[seg_v[s, pl.ds(16 * g, 16)] for g in range(D // 16)]
                       for s in range(N_SEG)]
                for j in range(16):
                    sj = siv[j]
                    w = l0 + j
                    lj = jnp.where(w >= L, w - L, w)
                    r = 16 * q + j
                    vals = [pe_v[lj, pl.ds(16 * g, 16)]
                            + jnp.where(sj == 1, sgv[1][g],
                                        jnp.where(sj >= 2, sgv[2][g], sgv[0][g]))
                            for g in range(D // 16)]
                    for g in range(D // 16):
                        plsc.addupdate(bufs[b].at[r, pl.ds(16 * g, 16)], vals[g])
                return 0

            lax.fori_loop(0, CH // 16, row_grp, 0)
            pltpu.async_copy(
                bufs[b], out_hbm.at[pl.ds(base + c * CH, CH)], sss[b])

        gather(0, 0)
        gather(1, 1)
        chunk(0, 0, "first")
        chunk(1, 1, "first")

        def quad_body(u, _):
            for j in range(NBUF):
                chunk(NBUF * u + 2 + j, (2 + j) % NBUF, "steady")
            return 0

        lax.fori_loop(0, (NCHUNK - 6) // NBUF, quad_body, 0)
        chunk(NCHUNK - 4, 2, "steady")
        chunk(NCHUNK - 3, 3, "steady")
        chunk(NCHUNK - 2, 0, "tail")
        chunk(NCHUNK - 1, 1, "tail")
        for c in range(NCHUNK - 4, NCHUNK):
            swait(c, c % NBUF)

    return k(token_table, pe2d, seg_table, tok_idx, seg_idx)


def _mask_body(seqt_ref, out_ref):
    m = (seqt_ref[...] > 0).astype(jnp.int8)
    out_ref[...] = jnp.broadcast_to(m[None, None, :, :], out_ref.shape)


def _tc_mask(seqt):
    Ib = 25
    maskt = pl.pallas_call(
        _mask_body,
        grid=(L // Ib,),
        in_specs=[pl.BlockSpec((L, B), lambda i: (0, 0))],
        out_specs=pl.BlockSpec((1, Ib, L, B), lambda i: (0, i, 0, 0)),
        out_shape=jax.ShapeDtypeStruct((1, L, L, B), jnp.int8),
    )(seqt)
    return jnp.transpose(maskt, (3, 0, 1, 2)).astype(jnp.bool_)


def kernel(input, token_table, segment_table, pe):
    seq = input[0]
    seg = input[1]
    tok_idx = seq.reshape(NW, NCHUNK, CH)
    seg_idx = seg.reshape(NW, NCHUNK, CH)
    x = _sc_embed(token_table, pe[0, :L], segment_table, tok_idx,
                  seg_idx).reshape(B, L, D)
    mask = _tc_mask(seq.T)
    return (x, mask)
